# Initial kernel scaffold; baseline (speedup 1.0000x reference)
import jax
import jax.numpy as jnp
from jax.experimental import pallas as pl


def kernel(outputs, label):
    raise NotImplementedError("write your pallas kernel here")


# speedup vs baseline: 396.9316x; 396.9316x over previous
import functools

import jax
import jax.numpy as jnp
from jax.experimental import pallas as pl
from jax.experimental.pallas import tpu as pltpu

_K = 15


def _tile_kernel(lab_ref, x_ref, out_ref, *, blk_r, cols):
    x = x_ref[...]
    lab = lab_ref[0, 0, :]
    col = jax.lax.broadcasted_iota(jnp.int32, (blk_r, cols), 1)
    eqlab = col == lab[:, None]
    xl = jnp.sum(jnp.where(eqlab, x, 0.0), axis=1)
    m = jnp.where(eqlab, jnp.float32(0.0), x)
    neg_inf = jnp.float32(-jnp.inf)

    v = jnp.max(m, axis=1)
    mx = jnp.maximum(jnp.maximum(v, xl), 0.0)
    s = jnp.exp(v - mx)
    for _ in range(1, _K):
        m = jnp.where(m == v[:, None], neg_inf, m)
        v = jnp.max(m, axis=1)
        s = s + jnp.exp(v - mx)
    s = s + jnp.exp(xl - mx) + (cols - _K - 1) * jnp.exp(-mx)
    lse = mx + jnp.log(s)
    lsm = xl - lse
    contrib = jnp.where(lsm == 0.0, jnp.float32(1e-10), lsm)
    out_ref[...] = (-jnp.sum(contrib)).reshape(1, 1, 1)


def kernel(outputs, label):
    rows, cols = outputs.shape
    blk_r = min(256, rows)
    nblk = rows // blk_r
    lab3 = label.reshape(nblk, 1, blk_r)
    partials = pl.pallas_call(
        functools.partial(_tile_kernel, blk_r=blk_r, cols=cols),
        grid=(nblk,),
        in_specs=[
            pl.BlockSpec((1, 1, blk_r), lambda i: (i, 0, 0)),
            pl.BlockSpec((blk_r, cols), lambda i: (i, 0)),
        ],
        out_specs=pl.BlockSpec((1, 1, 1), lambda i: (i, 0, 0)),
        out_shape=jax.ShapeDtypeStruct((nblk, 1, 1), jnp.float32),
        compiler_params=pltpu.CompilerParams(
            dimension_semantics=("arbitrary",),
        ),
    )(lab3, outputs)
    return jnp.sum(partials) / rows
